# Initial kernel scaffold; baseline (speedup 1.0000x reference)
import jax
import jax.numpy as jnp
from jax.experimental import pallas as pl


def kernel(node_feats, edge_index, Wq, bq, Wk, bk, Wv, bv):
    raise NotImplementedError("write your pallas kernel here")


# speedup vs baseline: 12.3948x; 12.3948x over previous
import functools

import jax
import jax.numpy as jnp
from jax import lax
from jax.experimental import pallas as pl
from jax.experimental.pallas import tpu as pltpu
from jax.experimental.pallas import tpu_sc as plsc

N = 10000
E = 320000
DIN = 128
H = 8
D = 16
HD = H * D
KVC = 2 * HD
ACC = HD + 16
NP = 10240
NCORES = 2
NSUB = 16
NW = NCORES * NSUB
EW = E // NW
CH = 40
NCH = EW // CH
RPS = NP // NSUB


def _proj_body(nf_ref, wq_ref, bq_ref, wk_ref, bk_ref, wv_ref, bv_ref,
               q_ref, kv_ref):
    nf = nf_ref[...]
    q_ref[...] = jnp.dot(nf, wq_ref[...],
                         preferred_element_type=jnp.float32) + bq_ref[...]
    k = (jnp.dot(nf, wk_ref[...],
                 preferred_element_type=jnp.float32) + bk_ref[...]) * 0.25
    v = jnp.dot(nf, wv_ref[...],
                preferred_element_type=jnp.float32) + bv_ref[...]
    kv_ref[:, :HD] = k
    kv_ref[:, HD:] = v


def _epi_body(p_ref, e_ref, o_ref):
    p = p_ref[0] + p_ref[1]
    zrep = jnp.dot(p, e_ref[...], preferred_element_type=jnp.float32)
    o_ref[...] = p[:, :HD] / zrep


_mesh = plsc.VectorSubcoreMesh(core_axis_name="c", subcore_axis_name="s")


@functools.partial(
    pl.kernel,
    mesh=_mesh,
    compiler_params=pltpu.CompilerParams(needs_layout_passes=False,
                                         use_tc_tiling_on_sc=False),
    out_type=jax.ShapeDtypeStruct((NCORES, NP, ACC), jnp.float32),
    scratch_types=[
        pltpu.VMEM((CH,), jnp.int32),
        pltpu.VMEM((CH,), jnp.int32),
        pltpu.VMEM((CH, KVC), jnp.float32),
        pltpu.VMEM((CH, HD), jnp.float32),
        pltpu.VMEM((CH, ACC), jnp.float32),
        pltpu.VMEM_SHARED((NP, ACC), jnp.float32),
    ],
)
def _sc_edge_kernel(kv_hbm, q_hbm, src_hbm, dst_hbm, out_hbm,
                    src_c, dst_c, kvg, qg, outc, acc_sh):
    cid = lax.axis_index("c")
    sid = lax.axis_index("s")
    wid = cid * NSUB + sid

    zero16 = jnp.zeros((16,), jnp.float32)

    def _zb(i, carry):
        for k in range(ACC // 16):
            outc[i, pl.ds(16 * k, 16)] = zero16
        return carry

    lax.fori_loop(0, CH, _zb, 0)
    base = sid * RPS

    def _zc(i, carry):
        pltpu.sync_copy(outc, acc_sh.at[pl.ds(base + CH * i, CH)])
        return carry

    lax.fori_loop(0, RPS // CH, _zc, 0)
    plsc.subcore_barrier()

    iota = lax.iota(jnp.int32, 16)
    masks = [jnp.where(iota == h, jnp.float32(1), jnp.float32(0))
             for h in range(H)]

    def _chunk(j, carry):
        pltpu.sync_copy(src_hbm.at[wid, j], src_c)
        pltpu.sync_copy(dst_hbm.at[wid, j], dst_c)
        pltpu.sync_copy(kv_hbm.at[src_c], kvg)
        pltpu.sync_copy(q_hbm.at[dst_c], qg)

        def _edge(e, c2):
            sv = jnp.zeros((16,), jnp.float32)
            for h in range(H):
                k = kvg[e, pl.ds(16 * h, 16)]
                q = qg[e, pl.ds(16 * h, 16)]
                s = jnp.sum(k * q)
                s = jnp.minimum(jnp.maximum(s, -5.0), 5.0)
                ev = jnp.exp(jnp.full((16,), s, jnp.float32))
                v = kvg[e, pl.ds(HD + 16 * h, 16)]
                outc[e, pl.ds(16 * h, 16)] = v * ev
                sv = sv + masks[h] * ev
            outc[e, pl.ds(HD, 16)] = sv
            return c2

        lax.fori_loop(0, CH, _edge, 0)
        pltpu.sync_copy(outc, acc_sh.at[dst_c], add=True)
        return carry

    lax.fori_loop(0, NCH, _chunk, 0)

    plsc.subcore_barrier()
    pltpu.sync_copy(acc_sh.at[pl.ds(base, RPS)],
                    out_hbm.at[cid, pl.ds(base, RPS)])


def kernel(node_feats, edge_index, Wq, bq, Wk, bk, Wv, bv):
    nf_pad = jnp.pad(node_feats, ((0, NP - N), (0, 0)))
    q_pad, kv_pad = pl.pallas_call(
        _proj_body,
        out_shape=(jax.ShapeDtypeStruct((NP, HD), jnp.float32),
                   jax.ShapeDtypeStruct((NP, KVC), jnp.float32)),
    )(nf_pad, Wq, bq.reshape(1, HD), Wk, bk.reshape(1, HD),
      Wv, bv.reshape(1, HD))

    src = edge_index[0].reshape(NW, NCH, CH)
    dst = edge_index[1].reshape(NW, NCH, CH)
    partial = _sc_edge_kernel(kv_pad, q_pad, src, dst)

    expand = jnp.zeros((ACC, HD), jnp.float32).at[HD:HD + H].set(
        jnp.repeat(jnp.eye(H, dtype=jnp.float32), D, axis=1))
    out = pl.pallas_call(
        _epi_body,
        out_shape=jax.ShapeDtypeStruct((NP, HD), jnp.float32),
    )(partial, expand)
    return out[:N].reshape(N, H, D)


# speedup vs baseline: 15.4184x; 1.2439x over previous
import functools

import jax
import jax.numpy as jnp
from jax import lax
from jax.experimental import pallas as pl
from jax.experimental.pallas import tpu as pltpu
from jax.experimental.pallas import tpu_sc as plsc

N = 10000
E = 320000
DIN = 128
H = 8
D = 16
HD = H * D
HH = H // 2
QC = HH * D
KV2 = 2 * QC
ACC = QC + 16
NSUB = 16
ET = E // NSUB
CH = 80
NCH = ET // CH
RPS = N // NSUB


def _proj_body(nf_ref, wq_ref, bq_ref, wk_ref, bk_ref, wv_ref, bv_ref,
               q_ref, kv_ref):
    nf = nf_ref[...]
    q = jnp.dot(nf, wq_ref[...], preferred_element_type=jnp.float32) + bq_ref[...]
    k = (jnp.dot(nf, wk_ref[...],
                 preferred_element_type=jnp.float32) + bk_ref[...]) * 0.25
    v = jnp.dot(nf, wv_ref[...], preferred_element_type=jnp.float32) + bv_ref[...]
    q_ref[0] = q[:, :QC]
    q_ref[1] = q[:, QC:]
    kv_ref[0] = jnp.concatenate([k[:, :QC], v[:, :QC]], axis=1)
    kv_ref[1] = jnp.concatenate([k[:, QC:], v[:, QC:]], axis=1)


def _epi_body(p_ref, e_ref, o_ref):
    p0 = p_ref[0]
    p1 = p_ref[1]
    z0 = jnp.dot(p0, e_ref[...], preferred_element_type=jnp.float32)
    z1 = jnp.dot(p1, e_ref[...], preferred_element_type=jnp.float32)
    o_ref[...] = jnp.concatenate([p0[:, :QC] / z0, p1[:, :QC] / z1], axis=1)


_mesh = plsc.VectorSubcoreMesh(core_axis_name="c", subcore_axis_name="s")


@functools.partial(
    pl.kernel,
    mesh=_mesh,
    compiler_params=pltpu.CompilerParams(needs_layout_passes=False,
                                         use_tc_tiling_on_sc=False),
    out_type=jax.ShapeDtypeStruct((2, N, ACC), jnp.float32),
    scratch_types=[
        pltpu.VMEM((NCH, CH), jnp.int32),
        pltpu.VMEM((NCH, CH), jnp.int32),
        pltpu.VMEM((CH, KV2), jnp.float32),
        pltpu.VMEM((CH, KV2), jnp.float32),
        pltpu.VMEM((CH, QC), jnp.float32),
        pltpu.VMEM((CH, QC), jnp.float32),
        pltpu.VMEM((CH, ACC), jnp.float32),
        pltpu.SemaphoreType.DMA,
        pltpu.SemaphoreType.DMA,
        pltpu.SemaphoreType.DMA,
        pltpu.SemaphoreType.DMA,
        pltpu.VMEM_SHARED((N, ACC), jnp.float32),
    ],
)
def _sc_edge_kernel(kv_hbm, q_hbm, src_hbm, dst_hbm, out_hbm,
                    src_v, dst_v, kvg0, kvg1, qg0, qg1, outc,
                    skv0, skv1, sq0, sq1, acc_sh):
    cid = lax.axis_index("c")
    sid = lax.axis_index("s")
    kv_t = kv_hbm.at[cid]
    q_t = q_hbm.at[cid]
    kvg = (kvg0, kvg1)
    qg = (qg0, qg1)
    skv = (skv0, skv1)
    sq = (sq0, sq1)

    pltpu.sync_copy(src_hbm.at[sid], src_v)
    pltpu.sync_copy(dst_hbm.at[sid], dst_v)

    pltpu.async_copy(kv_t.at[src_v.at[0]], kvg0, skv0)
    pltpu.async_copy(q_t.at[dst_v.at[0]], qg0, sq0)

    zero16 = jnp.zeros((16,), jnp.float32)

    def _zb(i, carry):
        for k in range(ACC // 16):
            outc[i, pl.ds(16 * k, 16)] = zero16
        return carry

    lax.fori_loop(0, CH, _zb, 0)
    base = sid * RPS

    def _zc(i, carry):
        pltpu.sync_copy(outc, acc_sh.at[pl.ds(base + CH * i, CH)])
        return carry

    lax.fori_loop(0, RPS // CH, _zc, 0)
    pltpu.sync_copy(outc.at[pl.ds(0, RPS - CH * (RPS // CH))],
                    acc_sh.at[pl.ds(base + CH * (RPS // CH),
                                    RPS - CH * (RPS // CH))])
    plsc.subcore_barrier()

    iota = lax.iota(jnp.int32, 16)
    masks = [jnp.where(iota == h, jnp.float32(1), jnp.float32(0))
             for h in range(HH)]

    def _pair(j2, carry):
        for b in range(2):
            j = 2 * j2 + b
            jn = jnp.where(j + 1 < NCH, j + 1, 0)
            pltpu.async_copy(kv_t.at[src_v.at[jn]], kvg[1 - b], skv[1 - b])
            pltpu.async_copy(q_t.at[dst_v.at[jn]], qg[1 - b], sq[1 - b])
            pltpu.make_async_copy(kv_t.at[src_v.at[j]], kvg[b], skv[b]).wait()
            pltpu.make_async_copy(q_t.at[dst_v.at[j]], qg[b], sq[b]).wait()

            def _edge(e, c2):
                sv = jnp.zeros((16,), jnp.float32)
                for h in range(HH):
                    k = kvg[b][e, pl.ds(16 * h, 16)]
                    q = qg[b][e, pl.ds(16 * h, 16)]
                    s = jnp.sum(k * q)
                    s = jnp.minimum(jnp.maximum(s, -5.0), 5.0)
                    ev = jnp.exp(jnp.full((16,), s, jnp.float32))
                    v = kvg[b][e, pl.ds(QC + 16 * h, 16)]
                    outc[e, pl.ds(16 * h, 16)] = v * ev
                    sv = sv + masks[h] * ev
                outc[e, pl.ds(QC, 16)] = sv
                return c2

            lax.fori_loop(0, CH, _edge, 0)
            pltpu.sync_copy(outc, acc_sh.at[dst_v.at[j]], add=True)
        return carry

    lax.fori_loop(0, NCH // 2, _pair, 0)
    pltpu.make_async_copy(kv_t.at[src_v.at[0]], kvg0, skv0).wait()
    pltpu.make_async_copy(q_t.at[dst_v.at[0]], qg0, sq0).wait()

    plsc.subcore_barrier()
    pltpu.sync_copy(acc_sh.at[pl.ds(base, RPS)],
                    out_hbm.at[cid, pl.ds(base, RPS)])


def kernel(node_feats, edge_index, Wq, bq, Wk, bk, Wv, bv):
    q2, kv2 = pl.pallas_call(
        _proj_body,
        out_shape=(jax.ShapeDtypeStruct((2, N, QC), jnp.float32),
                   jax.ShapeDtypeStruct((2, N, KV2), jnp.float32)),
    )(node_feats, Wq, bq.reshape(1, HD), Wk, bk.reshape(1, HD),
      Wv, bv.reshape(1, HD))

    src = edge_index[0].reshape(NSUB, NCH, CH)
    dst = edge_index[1].reshape(NSUB, NCH, CH)
    partial = _sc_edge_kernel(kv2, q2, src, dst)

    expand = jnp.zeros((ACC, QC), jnp.float32).at[QC:QC + HH].set(
        jnp.repeat(jnp.eye(HH, dtype=jnp.float32), D, axis=1))
    out = pl.pallas_call(
        _epi_body,
        out_shape=jax.ShapeDtypeStruct((N, HD), jnp.float32),
    )(partial, expand)
    return out.reshape(N, H, D)


# speedup vs baseline: 88.3508x; 5.7302x over previous
import functools

import jax
import jax.numpy as jnp
from jax import lax
from jax.experimental import pallas as pl
from jax.experimental.pallas import tpu as pltpu
from jax.experimental.pallas import tpu_sc as plsc

N = 10000
E = 320000
DIN = 128
H = 8
D = 16
HD = H * D
HH = H // 2
QC = HH * D
KV2 = 2 * QC
ACC = QC + 16
NSUB = 16
ET = E // NSUB
CH = 80
NCH = ET // CH
RPS = N // NSUB


def _proj_body(nf_ref, wq_ref, bq_ref, wk_ref, bk_ref, wv_ref, bv_ref,
               q_ref, kv_ref):
    nf = nf_ref[...]
    q = jnp.dot(nf, wq_ref[...], preferred_element_type=jnp.float32) + bq_ref[...]
    k = (jnp.dot(nf, wk_ref[...],
                 preferred_element_type=jnp.float32) + bk_ref[...]) * 0.25
    v = jnp.dot(nf, wv_ref[...], preferred_element_type=jnp.float32) + bv_ref[...]
    q_ref[0] = q[:, :QC]
    q_ref[1] = q[:, QC:]
    kv_ref[0] = jnp.concatenate([k[:, :QC], v[:, :QC]], axis=1)
    kv_ref[1] = jnp.concatenate([k[:, QC:], v[:, QC:]], axis=1)


def _epi_body(p_ref, e_ref, o_ref):
    p0 = p_ref[0]
    p1 = p_ref[1]
    z0 = jnp.dot(p0, e_ref[...], preferred_element_type=jnp.float32)
    z1 = jnp.dot(p1, e_ref[...], preferred_element_type=jnp.float32)
    o_ref[...] = jnp.concatenate([p0[:, :QC] / z0, p1[:, :QC] / z1], axis=1)


_mesh = plsc.VectorSubcoreMesh(core_axis_name="c", subcore_axis_name="s")


@functools.partial(
    pl.kernel,
    mesh=_mesh,
    compiler_params=pltpu.CompilerParams(needs_layout_passes=False,
                                         use_tc_tiling_on_sc=False),
    out_type=jax.ShapeDtypeStruct((2, N, ACC), jnp.float32),
    scratch_types=[
        pltpu.VMEM((NCH, CH), jnp.int32),
        pltpu.VMEM((NCH, CH), jnp.int32),
        pltpu.VMEM((CH, KV2), jnp.float32),
        pltpu.VMEM((CH, KV2), jnp.float32),
        pltpu.VMEM((CH, QC), jnp.float32),
        pltpu.VMEM((CH, QC), jnp.float32),
        pltpu.VMEM((CH, ACC), jnp.float32),
        pltpu.SemaphoreType.DMA,
        pltpu.SemaphoreType.DMA,
        pltpu.SemaphoreType.DMA,
        pltpu.SemaphoreType.DMA,
        pltpu.VMEM_SHARED((N, ACC), jnp.float32),
    ],
)
def _sc_edge_kernel(kv_hbm, q_hbm, src_hbm, dst_hbm, out_hbm,
                    src_v, dst_v, kvg0, kvg1, qg0, qg1, outc,
                    skv0, skv1, sq0, sq1, acc_sh):
    cid = lax.axis_index("c")
    sid = lax.axis_index("s")
    kv_t = kv_hbm.at[cid]
    q_t = q_hbm.at[cid]
    kvg = (kvg0, kvg1)
    qg = (qg0, qg1)
    skv = (skv0, skv1)
    sq = (sq0, sq1)

    pltpu.sync_copy(src_hbm.at[sid], src_v)
    pltpu.sync_copy(dst_hbm.at[sid], dst_v)

    pltpu.async_copy(kv_t.at[src_v.at[0]], kvg0, skv0)
    pltpu.async_copy(q_t.at[dst_v.at[0]], qg0, sq0)

    zero16 = jnp.zeros((16,), jnp.float32)

    def _zb(i, carry):
        for k in range(ACC // 16):
            outc[i, pl.ds(16 * k, 16)] = zero16
        return carry

    lax.fori_loop(0, CH, _zb, 0)
    base = sid * RPS

    def _zc(i, carry):
        pltpu.sync_copy(outc, acc_sh.at[pl.ds(base + CH * i, CH)])
        return carry

    lax.fori_loop(0, RPS // CH, _zc, 0)
    pltpu.sync_copy(outc.at[pl.ds(0, RPS - CH * (RPS // CH))],
                    acc_sh.at[pl.ds(base + CH * (RPS // CH),
                                    RPS - CH * (RPS // CH))])
    plsc.subcore_barrier()

    iota = lax.iota(jnp.int32, 16)
    masks = [jnp.where(iota == h, jnp.float32(1), jnp.float32(0))
             for h in range(HH)]
    mask0 = masks[0]

    def _pair(j2, carry):
        for b in range(2):
            j = 2 * j2 + b
            jn = jnp.where(j + 1 < NCH, j + 1, 0)
            pltpu.async_copy(kv_t.at[src_v.at[jn]], kvg[1 - b], skv[1 - b])
            pltpu.async_copy(q_t.at[dst_v.at[jn]], qg[1 - b], sq[1 - b])
            pltpu.make_async_copy(kv_t.at[src_v.at[j]], kvg[b], skv[b]).wait()
            pltpu.make_async_copy(q_t.at[dst_v.at[j]], qg[b], sq[b]).wait()

            @plsc.parallel_loop(0, CH, 1, unroll=4)
            def _edge(e):
                sv = jnp.zeros((16,), jnp.float32)
                for h in range(HH):
                    k = kvg[b][e, pl.ds(16 * h, 16)]
                    q = qg[b][e, pl.ds(16 * h, 16)]
                    c1 = plsc.cumsum(k * q)
                    bc = plsc.cumsum(jnp.flip(c1, axis=0) * mask0)
                    bc = jnp.minimum(jnp.maximum(bc, -5.0), 5.0)
                    ev = jnp.exp(bc)
                    v = kvg[b][e, pl.ds(QC + 16 * h, 16)]
                    outc[e, pl.ds(16 * h, 16)] = v * ev
                    sv = sv + masks[h] * ev
                outc[e, pl.ds(QC, 16)] = sv
            pltpu.sync_copy(outc, acc_sh.at[dst_v.at[j]], add=True)
        return carry

    lax.fori_loop(0, NCH // 2, _pair, 0)
    pltpu.make_async_copy(kv_t.at[src_v.at[0]], kvg0, skv0).wait()
    pltpu.make_async_copy(q_t.at[dst_v.at[0]], qg0, sq0).wait()

    plsc.subcore_barrier()
    pltpu.sync_copy(acc_sh.at[pl.ds(base, RPS)],
                    out_hbm.at[cid, pl.ds(base, RPS)])


def kernel(node_feats, edge_index, Wq, bq, Wk, bk, Wv, bv):
    q2, kv2 = pl.pallas_call(
        _proj_body,
        out_shape=(jax.ShapeDtypeStruct((2, N, QC), jnp.float32),
                   jax.ShapeDtypeStruct((2, N, KV2), jnp.float32)),
    )(node_feats, Wq, bq.reshape(1, HD), Wk, bk.reshape(1, HD),
      Wv, bv.reshape(1, HD))

    src = edge_index[0].reshape(NSUB, NCH, CH)
    dst = edge_index[1].reshape(NSUB, NCH, CH)
    partial = _sc_edge_kernel(kv2, q2, src, dst)

    expand = jnp.zeros((ACC, QC), jnp.float32).at[QC:QC + HH].set(
        jnp.repeat(jnp.eye(HH, dtype=jnp.float32), D, axis=1))
    out = pl.pallas_call(
        _epi_body,
        out_shape=jax.ShapeDtypeStruct((N, HD), jnp.float32),
    )(partial, expand)
    return out.reshape(N, H, D)
